# Initial kernel scaffold; baseline (speedup 1.0000x reference)
import functools

import jax
import jax.numpy as jnp
from jax import lax
from jax.experimental import pallas as pl
from jax.experimental.pallas import tpu as pltpu
from jax.experimental.pallas import tpu_sc as plsc

_NPOINT = 2048


def _fps_kernel(x_ref, y_ref, z_ref, f0_ref, idx_ref, xyz_ref, dist_ref,
                *, npoint):
    B, N = x_ref.shape
    x = x_ref[...]
    y = y_ref[...]
    z = z_ref[...]
    dist_ref[...] = jnp.full((B, N), 1e10, jnp.float32)
    iota = lax.broadcasted_iota(jnp.int32, (B, N), 1)
    boffs = lax.broadcasted_iota(jnp.int32, (1, B), 1) * N

    def body(i, f_col):
        mask = iota == f_col
        cx = jnp.sum(jnp.where(mask, x, 0.0), axis=1, keepdims=True)
        cy = jnp.sum(jnp.where(mask, y, 0.0), axis=1, keepdims=True)
        cz = jnp.sum(jnp.where(mask, z, 0.0), axis=1, keepdims=True)

        fbits = lax.bitcast_convert_type(f_col, jnp.float32)
        packed = jnp.concatenate([cx, cy, cz, fbits], axis=1)
        rows = jnp.transpose(packed)
        xyz_ref[pl.ds(i, 1), :, :] = rows[0:3].reshape(1, 3, B)
        f_row = lax.bitcast_convert_type(rows[3:4], jnp.int32)
        idx_ref[pl.ds(i, 1), :, :] = (f_row + boffs).reshape(1, 1, B)

        dx = x - cx
        dy = y - cy
        dz = z - cz
        d = jnp.minimum(dist_ref[...], dx * dx + dy * dy + dz * dz)
        dist_ref[...] = d
        m = jnp.max(d, axis=1, keepdims=True)
        cand = jnp.where(d == m, iota, N)
        return jnp.min(cand, axis=1, keepdims=True)

    lax.fori_loop(0, npoint, body, f0_ref[...])


def _fps(x, y, z, f0, npoint):
    B, N = x.shape
    idx_out, xyz_out = pl.pallas_call(
        functools.partial(_fps_kernel, npoint=npoint),
        out_shape=[
            jax.ShapeDtypeStruct((npoint, 1, B), jnp.int32),
            jax.ShapeDtypeStruct((npoint, 3, B), jnp.float32),
        ],
        scratch_shapes=[pltpu.VMEM((B, N), jnp.float32)],
    )(x, y, z, f0)
    return idx_out, xyz_out


def _sc_gather(table, idx):
    R, D = table.shape
    M = idx.shape[0]
    info = plsc.get_sparse_core_info()
    NW = info.num_cores * info.num_subcores
    b_per_w = M // NW
    CH = 128
    nch = b_per_w // CH
    mesh = plsc.VectorSubcoreMesh(core_axis_name="c", subcore_axis_name="s")

    @functools.partial(
        pl.kernel,
        mesh=mesh,
        out_type=jax.ShapeDtypeStruct((M, D), jnp.float32),
        scratch_types=[
            pltpu.VMEM((CH,), jnp.int32),
            pltpu.VMEM((CH, D), jnp.float32),
            pltpu.SemaphoreType.DMA,
        ],
    )
    def k(table_hbm, idx_hbm, out_hbm, idx_v, rows_v, sem):
        wid = lax.axis_index("s") * info.num_cores + lax.axis_index("c")
        base = wid * b_per_w

        def chunk(c, carry):
            off = pl.multiple_of(base + c * CH, CH)
            pltpu.sync_copy(idx_hbm.at[pl.ds(off, CH)], idx_v)
            pltpu.async_copy(table_hbm.at[idx_v], rows_v, sem).wait()
            pltpu.sync_copy(rows_v, out_hbm.at[pl.ds(off, CH)])
            return carry

        lax.fori_loop(0, nch, chunk, 0)

    return k(table, idx)


def _transpose_kernel(in_ref, out_ref):
    out_ref[0] = jnp.transpose(in_ref[0])


def _transpose_last2(g):
    B, S, D = g.shape
    TS = 512
    return pl.pallas_call(
        _transpose_kernel,
        grid=(B, S // TS),
        in_specs=[pl.BlockSpec((1, TS, D), lambda b, j: (b, j, 0))],
        out_specs=pl.BlockSpec((1, D, TS), lambda b, j: (b, 0, j)),
        out_shape=jax.ShapeDtypeStruct((B, D, S), jnp.float32),
    )(g)


def kernel(xyz, features):
    B, N, _ = xyz.shape
    D = features.shape[-1]
    S = _NPOINT

    planes = jnp.transpose(xyz, (2, 0, 1))
    f0 = jax.random.randint(
        jax.random.key(42), (B,), 0, N, dtype=jnp.int32
    ).reshape(B, 1)

    idx_out, xyz_out = _fps(planes[0], planes[1], planes[2], f0, S)
    new_xyz = jnp.transpose(xyz_out, (2, 1, 0))
    gidx = jnp.transpose(idx_out.reshape(S, B)).reshape(-1)

    gathered = _sc_gather(features.reshape(B * N, D), gidx)
    new_features = _transpose_last2(gathered.reshape(B, S, D))
    return new_xyz, new_features


# speedup vs baseline: 28.7810x; 28.7810x over previous
import functools

import jax
import jax.numpy as jnp
from jax import lax
from jax.experimental import pallas as pl
from jax.experimental.pallas import tpu as pltpu
from jax.experimental.pallas import tpu_sc as plsc

_NPOINT = 2048


def _fps_kernel(x_ref, y_ref, z_ref, f0_ref, idx_ref, xyz_ref, dist_ref,
                *, npoint):
    B, N = x_ref.shape
    x = x_ref[...]
    y = y_ref[...]
    z = z_ref[...]
    dist_ref[...] = jnp.full((B, N), 1e10, jnp.float32)
    iota = lax.broadcasted_iota(jnp.int32, (B, N), 1)
    boffs = lax.broadcasted_iota(jnp.int32, (1, B), 1) * N

    def body(i, f_col):
        mask = iota == f_col
        cx = jnp.sum(jnp.where(mask, x, 0.0), axis=1, keepdims=True)
        cy = jnp.sum(jnp.where(mask, y, 0.0), axis=1, keepdims=True)
        cz = jnp.sum(jnp.where(mask, z, 0.0), axis=1, keepdims=True)

        fbits = lax.bitcast_convert_type(f_col, jnp.float32)
        packed = jnp.concatenate([cx, cy, cz, fbits], axis=1)
        rows = jnp.transpose(packed)
        xyz_ref[pl.ds(i, 1), :, :] = rows[0:3].reshape(1, 3, B)
        f_row = lax.bitcast_convert_type(rows[3:4], jnp.int32)
        idx_ref[pl.ds(i, 1), :, :] = (f_row + boffs).reshape(1, 1, B)

        dx = x - cx
        dy = y - cy
        dz = z - cz
        d = jnp.minimum(dist_ref[...], (dx * dx + dz * dz) + dy * dy)
        dist_ref[...] = d
        m = jnp.max(d, axis=1, keepdims=True)
        cand = jnp.where(d == m, iota, N)
        return jnp.min(cand, axis=1, keepdims=True)

    lax.fori_loop(0, npoint, body, f0_ref[...])


def _fps(x, y, z, f0, npoint):
    B, N = x.shape
    idx_out, xyz_out = pl.pallas_call(
        functools.partial(_fps_kernel, npoint=npoint),
        out_shape=[
            jax.ShapeDtypeStruct((npoint, 1, B), jnp.int32),
            jax.ShapeDtypeStruct((npoint, 3, B), jnp.float32),
        ],
        scratch_shapes=[pltpu.VMEM((B, N), jnp.float32)],
    )(x, y, z, f0)
    return idx_out, xyz_out


def _sc_gather(table, idx):
    R, D = table.shape
    M = idx.shape[0]
    info = plsc.get_sparse_core_info()
    NW = info.num_cores * info.num_subcores
    b_per_w = M // NW
    CH = 128
    nch = b_per_w // CH
    mesh = plsc.VectorSubcoreMesh(core_axis_name="c", subcore_axis_name="s")

    @functools.partial(
        pl.kernel,
        mesh=mesh,
        out_type=jax.ShapeDtypeStruct((M, D), jnp.float32),
        scratch_types=[
            pltpu.VMEM((CH,), jnp.int32),
            pltpu.VMEM((CH, D), jnp.float32),
            pltpu.SemaphoreType.DMA,
        ],
    )
    def k(table_hbm, idx_hbm, out_hbm, idx_v, rows_v, sem):
        wid = lax.axis_index("s") * info.num_cores + lax.axis_index("c")
        base = wid * b_per_w

        def chunk(c, carry):
            off = pl.multiple_of(base + c * CH, CH)
            pltpu.sync_copy(idx_hbm.at[pl.ds(off, CH)], idx_v)
            pltpu.async_copy(table_hbm.at[idx_v], rows_v, sem).wait()
            pltpu.sync_copy(rows_v, out_hbm.at[pl.ds(off, CH)])
            return carry

        lax.fori_loop(0, nch, chunk, 0)

    return k(table, idx)


def _transpose_kernel(in_ref, out_ref):
    out_ref[0] = jnp.transpose(in_ref[0])


def _transpose_last2(g):
    B, S, D = g.shape
    TS = 512
    return pl.pallas_call(
        _transpose_kernel,
        grid=(B, S // TS),
        in_specs=[pl.BlockSpec((1, TS, D), lambda b, j: (b, j, 0))],
        out_specs=pl.BlockSpec((1, D, TS), lambda b, j: (b, 0, j)),
        out_shape=jax.ShapeDtypeStruct((B, D, S), jnp.float32),
    )(g)


def kernel(xyz, features):
    B, N, _ = xyz.shape
    D = features.shape[-1]
    S = _NPOINT

    planes = jnp.transpose(xyz, (2, 0, 1))
    f0 = jax.random.randint(
        jax.random.key(42), (B,), 0, N, dtype=jnp.int32
    ).reshape(B, 1)

    idx_out, xyz_out = _fps(planes[0], planes[1], planes[2], f0, S)
    new_xyz = jnp.transpose(xyz_out, (2, 1, 0))
    gidx = jnp.transpose(idx_out.reshape(S, B)).reshape(-1)

    gathered = _sc_gather(features.reshape(B * N, D), gidx)
    new_features = _transpose_last2(gathered.reshape(B, S, D))
    return new_xyz, new_features


# speedup vs baseline: 42.4132x; 1.4737x over previous
import functools

import jax
import jax.numpy as jnp
from jax import lax
from jax.experimental import pallas as pl
from jax.experimental.pallas import tpu as pltpu
from jax.experimental.pallas import tpu_sc as plsc

_NPOINT = 2048


_CW = 256


def _fps_kernel(x_ref, y_ref, z_ref, f0_ref, idx_ref, xyz_ref, dist_ref,
                iota_ref, *, npoint):
    B, N = x_ref.shape
    CW = _CW
    NCH = N // CW
    dist_ref[...] = jnp.full((B, N), 1e10, jnp.float32)
    iota_ref[...] = lax.broadcasted_iota(jnp.int32, (B, N), 1)
    boffs = lax.broadcasted_iota(jnp.int32, (1, B), 1) * N

    f0 = f0_ref[...]
    mask0 = iota_ref[...] == f0
    cx0 = jnp.sum(jnp.where(mask0, x_ref[...], 0.0), axis=1, keepdims=True)
    cy0 = jnp.sum(jnp.where(mask0, y_ref[...], 0.0), axis=1, keepdims=True)
    cz0 = jnp.sum(jnp.where(mask0, z_ref[...], 0.0), axis=1, keepdims=True)

    def body(i, carry):
        f_col, cx, cy, cz = carry

        fbits = lax.bitcast_convert_type(f_col, jnp.float32)
        packed = jnp.concatenate([cx, cy, cz, fbits], axis=1)
        rows = jnp.transpose(packed)
        xyz_ref[pl.ds(i, 1), :, :] = rows[0:3].reshape(1, 3, B)
        f_row = lax.bitcast_convert_type(rows[3:4], jnp.int32)
        idx_ref[pl.ds(i, 1), :, :] = (f_row + boffs).reshape(1, 1, B)

        runV = jnp.full((B, CW), -1.0, jnp.float32)
        runI = jnp.zeros((B, CW), jnp.int32)
        runX = jnp.zeros((B, CW), jnp.float32)
        runY = jnp.zeros((B, CW), jnp.float32)
        runZ = jnp.zeros((B, CW), jnp.float32)
        for j in range(NCH):
            sl = pl.ds(j * CW, CW)
            xj = x_ref[:, sl]
            yj = y_ref[:, sl]
            zj = z_ref[:, sl]
            dx = xj - cx
            dy = yj - cy
            dz = zj - cz
            dn = jnp.minimum(dist_ref[:, sl], (dx * dx + dz * dz) + dy * dy)
            dist_ref[:, sl] = dn
            gt = dn > runV
            runV = jnp.where(gt, dn, runV)
            runI = jnp.where(gt, iota_ref[:, sl], runI)
            runX = jnp.where(gt, xj, runX)
            runY = jnp.where(gt, yj, runY)
            runZ = jnp.where(gt, zj, runZ)

        m = jnp.max(runV, axis=1, keepdims=True)
        selm = runV == m
        f_new = jnp.min(jnp.where(selm, runI, N), axis=1, keepdims=True)
        am = runI == f_new
        ncx = jnp.sum(jnp.where(am, runX, 0.0), axis=1, keepdims=True)
        ncy = jnp.sum(jnp.where(am, runY, 0.0), axis=1, keepdims=True)
        ncz = jnp.sum(jnp.where(am, runZ, 0.0), axis=1, keepdims=True)
        return f_new, ncx, ncy, ncz

    lax.fori_loop(0, npoint, body, (f0, cx0, cy0, cz0))


def _fps(x, y, z, f0, npoint):
    B, N = x.shape
    idx_out, xyz_out = pl.pallas_call(
        functools.partial(_fps_kernel, npoint=npoint),
        out_shape=[
            jax.ShapeDtypeStruct((npoint, 1, B), jnp.int32),
            jax.ShapeDtypeStruct((npoint, 3, B), jnp.float32),
        ],
        scratch_shapes=[
            pltpu.VMEM((B, N), jnp.float32),
            pltpu.VMEM((B, N), jnp.int32),
        ],
    )(x, y, z, f0)
    return idx_out, xyz_out


def _sc_gather(table, idx):
    R, D = table.shape
    M = idx.shape[0]
    info = plsc.get_sparse_core_info()
    NW = info.num_cores * info.num_subcores
    b_per_w = M // NW
    CH = 128
    nch = b_per_w // CH
    mesh = plsc.VectorSubcoreMesh(core_axis_name="c", subcore_axis_name="s")

    @functools.partial(
        pl.kernel,
        mesh=mesh,
        out_type=jax.ShapeDtypeStruct((M, D), jnp.float32),
        scratch_types=[
            pltpu.VMEM((CH,), jnp.int32),
            pltpu.VMEM((CH, D), jnp.float32),
            pltpu.SemaphoreType.DMA,
        ],
    )
    def k(table_hbm, idx_hbm, out_hbm, idx_v, rows_v, sem):
        wid = lax.axis_index("s") * info.num_cores + lax.axis_index("c")
        base = wid * b_per_w

        def chunk(c, carry):
            off = pl.multiple_of(base + c * CH, CH)
            pltpu.sync_copy(idx_hbm.at[pl.ds(off, CH)], idx_v)
            pltpu.async_copy(table_hbm.at[idx_v], rows_v, sem).wait()
            pltpu.sync_copy(rows_v, out_hbm.at[pl.ds(off, CH)])
            return carry

        lax.fori_loop(0, nch, chunk, 0)

    return k(table, idx)


def _transpose_kernel(in_ref, out_ref):
    out_ref[0] = jnp.transpose(in_ref[0])


def _transpose_last2(g):
    B, S, D = g.shape
    TS = 512
    return pl.pallas_call(
        _transpose_kernel,
        grid=(B, S // TS),
        in_specs=[pl.BlockSpec((1, TS, D), lambda b, j: (b, j, 0))],
        out_specs=pl.BlockSpec((1, D, TS), lambda b, j: (b, 0, j)),
        out_shape=jax.ShapeDtypeStruct((B, D, S), jnp.float32),
    )(g)


def kernel(xyz, features):
    B, N, _ = xyz.shape
    D = features.shape[-1]
    S = _NPOINT

    planes = jnp.transpose(xyz, (2, 0, 1))
    f0 = jax.random.randint(
        jax.random.key(42), (B,), 0, N, dtype=jnp.int32
    ).reshape(B, 1)

    idx_out, xyz_out = _fps(planes[0], planes[1], planes[2], f0, S)
    new_xyz = jnp.transpose(xyz_out, (2, 1, 0))
    gidx = jnp.transpose(idx_out.reshape(S, B)).reshape(-1)

    gathered = _sc_gather(features.reshape(B * N, D), gidx)
    new_features = _transpose_last2(gathered.reshape(B, S, D))
    return new_xyz, new_features
